# baseline (speedup 1.0000x reference)
import functools

import jax
import jax.numpy as jnp
from jax.experimental import pallas as pl
from jax.experimental.pallas import tpu as pltpu


def _vq_body(idx_ref, prec_ref, ze_ref, books_ref, prob_ref,
             logp_ref, zq_ref):
    prec = prec_ref[0]
    i = pl.program_id(0)
    for h in range(2):
        ze = ze_ref[h]
        book = books_ref[idx_ref[2 * i + h]]
        zs = (2.0 * prec) * ze
        cross = jax.lax.dot_general(
            zs, book, (((1,), (1,)), ((), ())),
            preferred_element_type=jnp.float32)
        b_sq = jnp.sum(book * book, axis=1)
        t = cross - prec * b_sq[None, :]
        m = jnp.max(t, axis=1, keepdims=True)
        sh = t - m
        e = jnp.exp(sh)
        s = jnp.sum(e, axis=1, keepdims=True)
        prob_ref[h] = e / s
        logp_ref[h] = sh - jnp.log(s)
        iota = jax.lax.broadcasted_iota(jnp.int32, t.shape, 1)
        K = t.shape[1]
        cand = jnp.where(t == m, iota, K)
        am = jnp.min(cand, axis=1)
        enc = (iota == am[:, None]).astype(jnp.bfloat16)
        zq_ref[h] = jax.lax.dot_general(
            enc, book.astype(jnp.bfloat16), (((1,), (0,)), ((), ())),
            preferred_element_type=jnp.float32)


@jax.jit
def _vq(ze, c_logits, books, log_param_q):
    b, n, d = ze.shape
    n_books, K, _ = books.shape
    param_q = 1.0 + jnp.exp(log_param_q)
    precision_q = 0.5 / jnp.clip(param_q, 1e-10)
    idx = jnp.argmax(c_logits, axis=-1).astype(jnp.int32)
    prec_arr = jnp.reshape(precision_q.astype(jnp.float32), (1,))

    grid_spec = pltpu.PrefetchScalarGridSpec(
        num_scalar_prefetch=2,
        grid=(b // 2,),
        in_specs=[
            pl.BlockSpec((2, n, d), lambda i, idx, prec: (i, 0, 0)),
            pl.BlockSpec((n_books, K, d), lambda i, idx, prec: (0, 0, 0)),
        ],
        out_specs=[
            pl.BlockSpec((2, n, K), lambda i, idx, prec: (i, 0, 0)),
            pl.BlockSpec((2, n, K), lambda i, idx, prec: (i, 0, 0)),
            pl.BlockSpec((2, n, d), lambda i, idx, prec: (i, 0, 0)),
        ],
    )
    prob, log_prob, zq = pl.pallas_call(
        _vq_body,
        grid_spec=grid_spec,
        out_shape=[
            jax.ShapeDtypeStruct((b, n, K), jnp.float32),
            jax.ShapeDtypeStruct((b, n, K), jnp.float32),
            jax.ShapeDtypeStruct((b, n, d), jnp.float32),
        ],
    )(idx, prec_arr, ze, books)
    return zq, precision_q, prob, log_prob


def kernel(ze, c_logits, books, log_param_q, is_train):
    del is_train
    return _vq(ze, c_logits, books, log_param_q)


# speedup vs baseline: 1.0431x; 1.0431x over previous
import functools

import jax
import jax.numpy as jnp
from jax.experimental import pallas as pl
from jax.experimental.pallas import tpu as pltpu


def _vq_body(idx_ref, prec_ref, ze_ref, booka_ref, bookb_ref, prob_ref,
             logp_ref, zq_ref):
    prec = prec_ref[0]
    for h, book_ref in enumerate((booka_ref, bookb_ref)):
        ze = ze_ref[h]
        book = book_ref[0]
        zs = (2.0 * prec) * ze
        cross = jax.lax.dot_general(
            zs, book, (((1,), (1,)), ((), ())),
            preferred_element_type=jnp.float32)
        b_sq = jnp.sum(book * book, axis=1)
        t = cross - prec * b_sq[None, :]
        m = jnp.max(t, axis=1, keepdims=True)
        sh = t - m
        e = jnp.exp(sh)
        s = jnp.sum(e, axis=1, keepdims=True)
        prob_ref[h] = e / s
        logp_ref[h] = sh - jnp.log(s)
        iota = jax.lax.broadcasted_iota(
            jnp.int32, t.shape, 1).astype(jnp.float32)
        K = t.shape[1]
        cand = jnp.where(t == m, iota, float(K))
        am = jnp.min(cand, axis=1, keepdims=True)
        enc = (iota == am).astype(jnp.bfloat16)
        zq_ref[h] = jax.lax.dot_general(
            enc, book.astype(jnp.bfloat16), (((1,), (0,)), ((), ())),
            preferred_element_type=jnp.float32)


@jax.jit
def _vq(ze, c_logits, books, log_param_q):
    b, n, d = ze.shape
    n_books, K, _ = books.shape
    param_q = 1.0 + jnp.exp(log_param_q)
    precision_q = 0.5 / jnp.clip(param_q, 1e-10)
    idx = jnp.argmax(c_logits, axis=-1).astype(jnp.int32)
    prec_arr = jnp.reshape(precision_q.astype(jnp.float32), (1,))

    grid_spec = pltpu.PrefetchScalarGridSpec(
        num_scalar_prefetch=2,
        grid=(b // 2,),
        in_specs=[
            pl.BlockSpec((2, n, d), lambda i, idx, prec: (i, 0, 0)),
            pl.BlockSpec((1, K, d), lambda i, idx, prec: (idx[2 * i], 0, 0)),
            pl.BlockSpec((1, K, d), lambda i, idx, prec: (idx[2 * i + 1], 0, 0)),
        ],
        out_specs=[
            pl.BlockSpec((2, n, K), lambda i, idx, prec: (i, 0, 0)),
            pl.BlockSpec((2, n, K), lambda i, idx, prec: (i, 0, 0)),
            pl.BlockSpec((2, n, d), lambda i, idx, prec: (i, 0, 0)),
        ],
    )
    prob, log_prob, zq = pl.pallas_call(
        _vq_body,
        grid_spec=grid_spec,
        out_shape=[
            jax.ShapeDtypeStruct((b, n, K), jnp.float32),
            jax.ShapeDtypeStruct((b, n, K), jnp.float32),
            jax.ShapeDtypeStruct((b, n, d), jnp.float32),
        ],
    )(idx, prec_arr, ze, books, books)
    return zq, precision_q, prob, log_prob


def kernel(ze, c_logits, books, log_param_q, is_train):
    del is_train
    return _vq(ze, c_logits, books, log_param_q)
